# Initial kernel scaffold; baseline (speedup 1.0000x reference)
import jax
import jax.numpy as jnp
from jax.experimental import pallas as pl


def kernel(x, edge_index, graph_ids, W1, b1, W2, b2, Wf1, bf1, Wf2, bf2):
    raise NotImplementedError("write your pallas kernel here")


# speedup vs baseline: 27.7387x; 27.7387x over previous
import functools

import jax
import jax.numpy as jnp
from jax import lax
from jax.experimental import pallas as pl
from jax.experimental.pallas import tpu as pltpu
from jax.experimental.pallas import tpu_sc as plsc

N = 100000
F = 16
NGRAPH = 64
BLK = 2000
NBLK = N // BLK
EROWS = 25000
RPT = 776
KB = 8
NCHUNK = 97
WCHUNK = 6256
ZC = 784


def _edge_pass(with_deg):
    mesh = plsc.VectorSubcoreMesh(core_axis_name="c", subcore_axis_name="s")
    out_type = [jax.ShapeDtypeStruct((2, N, F), jnp.float32)]
    scratch = [
        pltpu.VMEM_SHARED((N, F), jnp.float32),
        pltpu.VMEM((KB, 128), jnp.int32),
        pltpu.VMEM((KB, 128), jnp.int32),
        pltpu.VMEM((KB * 128, F), jnp.float32),
        pltpu.SemaphoreType.DMA,
    ]
    if with_deg:
        out_type.append(jax.ShapeDtypeStruct((2, N), jnp.float32))
        scratch += [
            pltpu.VMEM_SHARED((N,), jnp.float32),
            pltpu.VMEM((128,), jnp.float32),
            pltpu.VMEM((640,), jnp.float32),
        ]

    def body(*refs):
        if with_deg:
            (src_hbm, dst_hbm, z_hbm, acc_out, deg_out,
             acc_sh, sidx, didx, rows, sem, deg_sh, ones_v, dzero) = refs
        else:
            (src_hbm, dst_hbm, z_hbm, acc_out,
             acc_sh, sidx, didx, rows, sem) = refs
        c = lax.axis_index("c")
        s = lax.axis_index("s")
        wid = c * 16 + s

        def zrow(i, carry):
            rows[i, :] = jnp.zeros((F,), jnp.float32)
            return carry
        lax.fori_loop(0, KB * 128, zrow, 0)

        def zacc(k, carry):
            pltpu.sync_copy(rows,
                            acc_sh.at[pl.ds(s * WCHUNK + k * (KB * 128),
                                            KB * 128)])
            return carry
        lax.fori_loop(0, 6, zacc, 0)

        @pl.when(s < 15)
        def _():
            pltpu.sync_copy(rows.at[pl.ds(0, WCHUNK - 6144)],
                            acc_sh.at[pl.ds(s * WCHUNK + 6144,
                                            WCHUNK - 6144)])

        @pl.when(s == 15)
        def _():
            pltpu.sync_copy(rows.at[pl.ds(0, N - 15 * WCHUNK - 6144)],
                            acc_sh.at[pl.ds(15 * WCHUNK + 6144,
                                            N - 15 * WCHUNK - 6144)])

        if with_deg:
            def o16(i, carry):
                ones_v[pl.ds(i * 16, 16)] = jnp.ones((16,), jnp.float32)
                return carry
            lax.fori_loop(0, 8, o16, 0)

            @pl.when(s == 0)
            def _():
                def dz(i, carry):
                    dzero[pl.ds(i * 16, 16)] = jnp.zeros((16,), jnp.float32)
                    return carry
                lax.fori_loop(0, 40, dz, 0)

                def dzc(k, carry):
                    pltpu.sync_copy(dzero, deg_sh.at[pl.ds(k * 640, 640)])
                    return carry
                lax.fori_loop(0, 156, dzc, 0)
                pltpu.sync_copy(dzero.at[pl.ds(0, 160)],
                                deg_sh.at[pl.ds(156 * 640, 160)])

        plsc.subcore_barrier()

        def process(rowbase):
            pltpu.sync_copy(src_hbm.at[pl.ds(rowbase, KB)], sidx)
            pltpu.sync_copy(dst_hbm.at[pl.ds(rowbase, KB)], didx)
            cps = [pltpu.async_copy(z_hbm.at[sidx.at[j]],
                                    rows.at[pl.ds(j * 128, 128)], sem)
                   for j in range(KB)]
            for cp in cps:
                cp.wait()
            for j in range(KB):
                pltpu.sync_copy(rows.at[pl.ds(j * 128, 128)],
                                acc_sh.at[didx.at[j]], add=True)
                if with_deg:
                    pltpu.sync_copy(ones_v, deg_sh.at[didx.at[j]], add=True)

        def chunk(it, carry):
            process(wid * RPT + it * KB)
            return carry
        lax.fori_loop(0, NCHUNK, chunk, 0)

        @pl.when(wid < (EROWS - 32 * RPT) // KB)
        def _():
            process(32 * RPT + wid * KB)

        plsc.subcore_barrier()

        def writeout(dst_view, src_view):
            @pl.when(s < 15)
            def _():
                pltpu.sync_copy(src_view.at[pl.ds(s * WCHUNK, WCHUNK)],
                                dst_view.at[pl.ds(s * WCHUNK, WCHUNK)])

            @pl.when(s == 15)
            def _():
                pltpu.sync_copy(
                    src_view.at[pl.ds(15 * WCHUNK, N - 15 * WCHUNK)],
                    dst_view.at[pl.ds(15 * WCHUNK, N - 15 * WCHUNK)])

        writeout(acc_out.at[c], acc_sh)
        if with_deg:
            @pl.when(s < 15)
            def _():
                pltpu.sync_copy(deg_sh.at[pl.ds(s * 6656, 6656)],
                                deg_out.at[c].at[pl.ds(s * 6656, 6656)])

            @pl.when(s == 15)
            def _():
                pltpu.sync_copy(deg_sh.at[pl.ds(15 * 6656, N - 15 * 6656)],
                                deg_out.at[c].at[pl.ds(15 * 6656,
                                                       N - 15 * 6656)])

    return pl.kernel(
        body, mesh=mesh, out_type=out_type, scratch_types=scratch,
        compiler_params=pltpu.CompilerParams(use_tc_tiling_on_sc=False))


_edge_pass_deg = _edge_pass(True)
_edge_pass_nodeg = _edge_pass(False)


def _z1_body(x_ref, w_ref, o_ref):
    o_ref[...] = jnp.dot(x_ref[...], w_ref[...],
                         preferred_element_type=jnp.float32)


def _mid_body(z_ref, a0_ref, a1_ref, d0_ref, d1_ref, b_ref, w_ref, o_ref):
    acc = a0_ref[...] + a1_ref[...]
    deg = d0_ref[0] + d1_ref[0]
    mean = acc / jnp.maximum(deg, 1.0)
    h = jnp.where(deg > 0, mean, z_ref[...]) + b_ref[...]
    h = jnp.maximum(h, 0.0)
    o_ref[...] = jnp.dot(h, w_ref[...], preferred_element_type=jnp.float32)


def _fin_body(z_ref, a0_ref, a1_ref, d0_ref, d1_ref, b_ref, g_ref,
              wf1_ref, bf1_ref, wf2_ref, bf2_ref, o_ref, hg_scr, cnt_scr):
    i = pl.program_id(0)

    @pl.when(i == 0)
    def _():
        hg_scr[...] = jnp.zeros_like(hg_scr)
        cnt_scr[...] = jnp.zeros_like(cnt_scr)

    acc = a0_ref[...] + a1_ref[...]
    deg = d0_ref[0] + d1_ref[0]
    mean = acc / jnp.maximum(deg, 1.0)
    h = jnp.where(deg > 0, mean, z_ref[...]) + b_ref[...]
    h = jnp.maximum(h, 0.0)
    g = g_ref[0]
    gio = lax.broadcasted_iota(jnp.int32, (NGRAPH, BLK), 0)
    oh = (g == gio).astype(jnp.float32)
    hg_scr[...] += jnp.dot(oh, h, preferred_element_type=jnp.float32)
    cnt_scr[...] += jnp.sum(oh, axis=1, keepdims=True)

    @pl.when(i == NBLK - 1)
    def _():
        hg = hg_scr[...] / jnp.maximum(cnt_scr[...], 1.0)
        a = jnp.dot(hg, wf1_ref[...],
                    preferred_element_type=jnp.float32) + bf1_ref[...]
        p = jnp.dot(a, wf2_ref[...],
                    preferred_element_type=jnp.float32) + bf2_ref[...]
        o_ref[...] = 1.0 / (1.0 + jnp.exp(-p))


def kernel(x, edge_index, graph_ids, W1, b1, W2, b2, Wf1, bf1, Wf2, bf2):
    src = edge_index[0].astype(jnp.int32).reshape(EROWS, 128)
    dst = edge_index[1].astype(jnp.int32).reshape(EROWS, 128)
    gid_r = graph_ids.astype(jnp.int32).reshape(NBLK, 1, BLK)

    z1 = pl.pallas_call(
        _z1_body,
        grid=(NBLK,),
        in_specs=[pl.BlockSpec((BLK, 20), lambda i: (i, 0)),
                  pl.BlockSpec((20, F), lambda i: (0, 0))],
        out_specs=pl.BlockSpec((BLK, F), lambda i: (i, 0)),
        out_shape=jax.ShapeDtypeStruct((N, F), jnp.float32),
    )(x, W1)

    acc1, deg = _edge_pass_deg(src, dst, z1)
    d0 = deg[0].reshape(NBLK, BLK, 1)
    d1 = deg[1].reshape(NBLK, BLK, 1)

    nf_spec = pl.BlockSpec((BLK, F), lambda i: (i, 0))
    dg_spec = pl.BlockSpec((1, BLK, 1), lambda i: (i, 0, 0))

    z2 = pl.pallas_call(
        _mid_body,
        grid=(NBLK,),
        in_specs=[nf_spec, nf_spec, nf_spec, dg_spec, dg_spec,
                  pl.BlockSpec((1, F), lambda i: (0, 0)),
                  pl.BlockSpec((F, F), lambda i: (0, 0))],
        out_specs=nf_spec,
        out_shape=jax.ShapeDtypeStruct((N, F), jnp.float32),
    )(z1, acc1[0], acc1[1], d0, d1, b1.reshape(1, F), W2)

    (acc2,) = _edge_pass_nodeg(src, dst, z2)

    out = pl.pallas_call(
        _fin_body,
        grid=(NBLK,),
        in_specs=[nf_spec, nf_spec, nf_spec, dg_spec, dg_spec,
                  pl.BlockSpec((1, F), lambda i: (0, 0)),
                  pl.BlockSpec((1, 1, BLK), lambda i: (i, 0, 0)),
                  pl.BlockSpec((F, 8), lambda i: (0, 0)),
                  pl.BlockSpec((1, 8), lambda i: (0, 0)),
                  pl.BlockSpec((8, 1), lambda i: (0, 0)),
                  pl.BlockSpec((1, 1), lambda i: (0, 0))],
        out_specs=pl.BlockSpec((NGRAPH, 1), lambda i: (0, 0)),
        out_shape=jax.ShapeDtypeStruct((NGRAPH, 1), jnp.float32),
        scratch_shapes=[pltpu.VMEM((NGRAPH, F), jnp.float32),
                        pltpu.VMEM((NGRAPH, 1), jnp.float32)],
    )(z2, acc2[0], acc2[1], d0, d1, b2.reshape(1, F), gid_r,
      Wf1, bf1.reshape(1, 8), Wf2, bf2.reshape(1, 1))

    return out.reshape(NGRAPH)


# speedup vs baseline: 33.8211x; 1.2193x over previous
import functools

import jax
import jax.numpy as jnp
from jax import lax
from jax.experimental import pallas as pl
from jax.experimental.pallas import tpu as pltpu
from jax.experimental.pallas import tpu_sc as plsc

N = 100000
F = 16
NGRAPH = 64
BLK = 2000
NBLK = N // BLK
EROWS = 25000
RPT = 776
KB = 8
NCHUNK = 97
WCHUNK = 6256
ZC = 784


def _edge_pass(with_deg):
    mesh = plsc.VectorSubcoreMesh(core_axis_name="c", subcore_axis_name="s")
    out_type = [jax.ShapeDtypeStruct((2, N, F), jnp.float32)]
    scratch = [
        pltpu.VMEM_SHARED((N, F), jnp.float32),
        pltpu.VMEM((KB, 128), jnp.int32),
        pltpu.VMEM((KB, 128), jnp.int32),
        pltpu.VMEM((KB * 128, F), jnp.float32),
        pltpu.SemaphoreType.DMA,
    ]
    if with_deg:
        out_type.append(jax.ShapeDtypeStruct((2, N), jnp.float32))
        scratch += [
            pltpu.VMEM_SHARED((N,), jnp.float32),
            pltpu.VMEM((128,), jnp.float32),
            pltpu.VMEM((640,), jnp.float32),
        ]

    def body(*refs):
        if with_deg:
            (src_hbm, dst_hbm, z_hbm, acc_out, deg_out,
             acc_sh, sidx, didx, rows, sem, deg_sh, ones_v, dzero) = refs
        else:
            (src_hbm, dst_hbm, z_hbm, acc_out,
             acc_sh, sidx, didx, rows, sem) = refs
        c = lax.axis_index("c")
        s = lax.axis_index("s")
        wid = c * 16 + s

        def zrow(i, carry):
            rows[i, :] = jnp.zeros((F,), jnp.float32)
            return carry
        lax.fori_loop(0, KB * 128, zrow, 0)

        def zacc(k, carry):
            pltpu.sync_copy(rows,
                            acc_sh.at[pl.ds(s * WCHUNK + k * (KB * 128),
                                            KB * 128)])
            return carry
        lax.fori_loop(0, 6, zacc, 0)

        @pl.when(s < 15)
        def _():
            pltpu.sync_copy(rows.at[pl.ds(0, WCHUNK - 6144)],
                            acc_sh.at[pl.ds(s * WCHUNK + 6144,
                                            WCHUNK - 6144)])

        @pl.when(s == 15)
        def _():
            pltpu.sync_copy(rows.at[pl.ds(0, N - 15 * WCHUNK - 6144)],
                            acc_sh.at[pl.ds(15 * WCHUNK + 6144,
                                            N - 15 * WCHUNK - 6144)])

        if with_deg:
            def o16(i, carry):
                ones_v[pl.ds(i * 16, 16)] = jnp.ones((16,), jnp.float32)
                return carry
            lax.fori_loop(0, 8, o16, 0)

            @pl.when(s == 0)
            def _():
                def dz(i, carry):
                    dzero[pl.ds(i * 16, 16)] = jnp.zeros((16,), jnp.float32)
                    return carry
                lax.fori_loop(0, 40, dz, 0)

                def dzc(k, carry):
                    pltpu.sync_copy(dzero, deg_sh.at[pl.ds(k * 640, 640)])
                    return carry
                lax.fori_loop(0, 156, dzc, 0)
                pltpu.sync_copy(dzero.at[pl.ds(0, 160)],
                                deg_sh.at[pl.ds(156 * 640, 160)])

        plsc.subcore_barrier()

        def process(rowbase):
            pltpu.sync_copy(src_hbm.at[pl.ds(rowbase, KB)], sidx)
            pltpu.sync_copy(dst_hbm.at[pl.ds(rowbase, KB)], didx)
            cps = [pltpu.async_copy(z_hbm.at[sidx.at[j]],
                                    rows.at[pl.ds(j * 128, 128)], sem)
                   for j in range(KB)]
            for cp in cps:
                cp.wait()
            for j in range(KB):
                pltpu.sync_copy(rows.at[pl.ds(j * 128, 128)],
                                acc_sh.at[didx.at[j]], add=True)
                if with_deg:
                    pltpu.sync_copy(ones_v, deg_sh.at[didx.at[j]], add=True)

        def chunk(it, carry):
            process(wid * RPT + it * KB)
            return carry
        lax.fori_loop(0, NCHUNK, chunk, 0)

        @pl.when(wid < (EROWS - 32 * RPT) // KB)
        def _():
            process(32 * RPT + wid * KB)

        plsc.subcore_barrier()

        def writeout(dst_view, src_view):
            @pl.when(s < 15)
            def _():
                pltpu.sync_copy(src_view.at[pl.ds(s * WCHUNK, WCHUNK)],
                                dst_view.at[pl.ds(s * WCHUNK, WCHUNK)])

            @pl.when(s == 15)
            def _():
                pltpu.sync_copy(
                    src_view.at[pl.ds(15 * WCHUNK, N - 15 * WCHUNK)],
                    dst_view.at[pl.ds(15 * WCHUNK, N - 15 * WCHUNK)])

        writeout(acc_out.at[c], acc_sh)
        if with_deg:
            @pl.when(s < 15)
            def _():
                pltpu.sync_copy(deg_sh.at[pl.ds(s * 6656, 6656)],
                                deg_out.at[c].at[pl.ds(s * 6656, 6656)])

            @pl.when(s == 15)
            def _():
                pltpu.sync_copy(deg_sh.at[pl.ds(15 * 6656, N - 15 * 6656)],
                                deg_out.at[c].at[pl.ds(15 * 6656,
                                                       N - 15 * 6656)])

    return pl.kernel(
        body, mesh=mesh, out_type=out_type, scratch_types=scratch,
        compiler_params=pltpu.CompilerParams(use_tc_tiling_on_sc=False))


_edge_pass_deg = _edge_pass(True)
_edge_pass_nodeg = _edge_pass(False)


def _z1_body(x_ref, w_ref, o_ref):
    o_ref[...] = jnp.dot(x_ref[...], w_ref[...],
                         preferred_element_type=jnp.float32)


def _eye():
    return (lax.broadcasted_iota(jnp.int32, (F, F), 0)
            == lax.broadcasted_iota(jnp.int32, (F, F), 1)).astype(jnp.float32)


def _dg(a, b, dims):
    return lax.dot_general(a, b, (dims, ((), ())),
                           preferred_element_type=jnp.float32)


def _hidden_t(z_ref, acc_ref, deg_ref, b_ref):
    eye = _eye()
    acc = acc_ref[0] + acc_ref[1]
    deg = deg_ref[0, 0, 0] + deg_ref[1, 0, 0]
    acc_t = _dg(eye, acc, ((1,), (1,)))
    z_t = _dg(eye, z_ref[...], ((1,), (1,)))
    mean_t = acc_t / jnp.maximum(deg, 1.0)
    h_t = jnp.where(deg > 0, mean_t, z_t) + b_ref[...]
    return jnp.maximum(h_t, 0.0)


def _mid_body(z_ref, acc_ref, deg_ref, b_ref, w_ref, o_ref):
    h_t = _hidden_t(z_ref, acc_ref, deg_ref, b_ref)
    o_ref[...] = _dg(h_t, w_ref[...], ((0,), (0,)))


def _fin_body(z_ref, acc_ref, deg_ref, b_ref, g_ref,
              wf1_ref, bf1_ref, wf2_ref, bf2_ref, o_ref, hg_scr, cnt_scr):
    i = pl.program_id(0)

    @pl.when(i == 0)
    def _():
        hg_scr[...] = jnp.zeros_like(hg_scr)
        cnt_scr[...] = jnp.zeros_like(cnt_scr)

    h_t = _hidden_t(z_ref, acc_ref, deg_ref, b_ref)
    g = g_ref[0, 0]
    gio = lax.broadcasted_iota(jnp.int32, (NGRAPH, BLK), 0)
    oh = (g == gio).astype(jnp.float32)
    hg_scr[...] += _dg(oh, h_t, ((1,), (1,)))
    cnt_scr[...] += jnp.sum(oh, axis=1, keepdims=True)

    @pl.when(i == NBLK - 1)
    def _():
        hg = hg_scr[...] / jnp.maximum(cnt_scr[...], 1.0)
        a = jnp.dot(hg, wf1_ref[...],
                    preferred_element_type=jnp.float32) + bf1_ref[...]
        p = jnp.dot(a, wf2_ref[...],
                    preferred_element_type=jnp.float32) + bf2_ref[...]
        o_ref[...] = 1.0 / (1.0 + jnp.exp(-p))


def kernel(x, edge_index, graph_ids, W1, b1, W2, b2, Wf1, bf1, Wf2, bf2):
    src = edge_index[0].astype(jnp.int32).reshape(EROWS, 128)
    dst = edge_index[1].astype(jnp.int32).reshape(EROWS, 128)
    gid = graph_ids.astype(jnp.int32)

    z1 = pl.pallas_call(
        _z1_body,
        grid=(NBLK,),
        in_specs=[pl.BlockSpec((BLK, 20), lambda i: (i, 0)),
                  pl.BlockSpec((20, F), lambda i: (0, 0))],
        out_specs=pl.BlockSpec((BLK, F), lambda i: (i, 0)),
        out_shape=jax.ShapeDtypeStruct((N, F), jnp.float32),
    )(x, W1)

    acc1, deg = _edge_pass_deg(src, dst, z1)
    deg4 = deg.reshape(2, NBLK, 1, BLK)

    nf_spec = pl.BlockSpec((BLK, F), lambda i: (i, 0))
    acc_spec = pl.BlockSpec((2, BLK, F), lambda i: (0, i, 0))
    deg_spec = pl.BlockSpec((2, 1, 1, BLK), lambda i: (0, i, 0, 0))
    bt_spec = pl.BlockSpec((F, 1), lambda i: (0, 0))

    z2 = pl.pallas_call(
        _mid_body,
        grid=(NBLK,),
        in_specs=[nf_spec, acc_spec, deg_spec, bt_spec,
                  pl.BlockSpec((F, F), lambda i: (0, 0))],
        out_specs=nf_spec,
        out_shape=jax.ShapeDtypeStruct((N, F), jnp.float32),
    )(z1, acc1, deg4, b1.reshape(F, 1), W2)

    (acc2,) = _edge_pass_nodeg(src, dst, z2)

    out = pl.pallas_call(
        _fin_body,
        grid=(NBLK,),
        in_specs=[nf_spec, acc_spec, deg_spec, bt_spec,
                  pl.BlockSpec((1, 1, BLK), lambda i: (i, 0, 0)),
                  pl.BlockSpec((F, 8), lambda i: (0, 0)),
                  pl.BlockSpec((1, 8), lambda i: (0, 0)),
                  pl.BlockSpec((8, 1), lambda i: (0, 0)),
                  pl.BlockSpec((1, 1), lambda i: (0, 0))],
        out_specs=pl.BlockSpec((NGRAPH, 1), lambda i: (0, 0)),
        out_shape=jax.ShapeDtypeStruct((NGRAPH, 1), jnp.float32),
        scratch_shapes=[pltpu.VMEM((NGRAPH, F), jnp.float32),
                        pltpu.VMEM((NGRAPH, 1), jnp.float32)],
    )(z2, acc2, deg4, b2.reshape(F, 1), gid.reshape(NBLK, 1, BLK),
      Wf1, bf1.reshape(1, 8), Wf2, bf2.reshape(1, 1))

    return out.reshape(NGRAPH)


# speedup vs baseline: 38.0314x; 1.1245x over previous
import functools

import jax
import jax.numpy as jnp
from jax import lax
from jax.experimental import pallas as pl
from jax.experimental.pallas import tpu as pltpu
from jax.experimental.pallas import tpu_sc as plsc

N = 100000
F = 16
NGRAPH = 64
BLK = 2000
NBLK = N // BLK
EROWS = 25000
RPT = 776
KB = 8
NCHUNK = 97
WCHUNK = 6256
ZC = 784


def _edge_pass(with_deg):
    mesh = plsc.VectorSubcoreMesh(core_axis_name="c", subcore_axis_name="s")
    out_type = [jax.ShapeDtypeStruct((2, N, F), jnp.float32)]
    scratch = [
        pltpu.VMEM_SHARED((N, F), jnp.float32),
        pltpu.VMEM((KB, 128), jnp.int32),
        pltpu.VMEM((KB, 128), jnp.int32),
        pltpu.VMEM((KB * 128, F), jnp.float32),
        pltpu.SemaphoreType.DMA,
    ]
    if with_deg:
        out_type.append(jax.ShapeDtypeStruct((2, N), jnp.float32))
        scratch += [
            pltpu.VMEM_SHARED((N,), jnp.float32),
            pltpu.VMEM((128,), jnp.float32),
            pltpu.VMEM((640,), jnp.float32),
        ]

    def body(*refs):
        if with_deg:
            (e_hbm, z_hbm, acc_out, deg_out,
             acc_sh, sidx, didx, rows, sem, deg_sh, ones_v, dzero) = refs
        else:
            (e_hbm, z_hbm, acc_out,
             acc_sh, sidx, didx, rows, sem) = refs
        src_hbm = e_hbm.at[0]
        dst_hbm = e_hbm.at[1]
        c = lax.axis_index("c")
        s = lax.axis_index("s")
        wid = c * 16 + s

        def zrow(i, carry):
            rows[i, :] = jnp.zeros((F,), jnp.float32)
            return carry
        lax.fori_loop(0, KB * 128, zrow, 0)

        def zacc(k, carry):
            pltpu.sync_copy(rows,
                            acc_sh.at[pl.ds(s * WCHUNK + k * (KB * 128),
                                            KB * 128)])
            return carry
        lax.fori_loop(0, 6, zacc, 0)

        @pl.when(s < 15)
        def _():
            pltpu.sync_copy(rows.at[pl.ds(0, WCHUNK - 6144)],
                            acc_sh.at[pl.ds(s * WCHUNK + 6144,
                                            WCHUNK - 6144)])

        @pl.when(s == 15)
        def _():
            pltpu.sync_copy(rows.at[pl.ds(0, N - 15 * WCHUNK - 6144)],
                            acc_sh.at[pl.ds(15 * WCHUNK + 6144,
                                            N - 15 * WCHUNK - 6144)])

        if with_deg:
            def o16(i, carry):
                ones_v[pl.ds(i * 16, 16)] = jnp.ones((16,), jnp.float32)
                return carry
            lax.fori_loop(0, 8, o16, 0)

            @pl.when(s == 0)
            def _():
                def dz(i, carry):
                    dzero[pl.ds(i * 16, 16)] = jnp.zeros((16,), jnp.float32)
                    return carry
                lax.fori_loop(0, 40, dz, 0)

                def dzc(k, carry):
                    pltpu.sync_copy(dzero, deg_sh.at[pl.ds(k * 640, 640)])
                    return carry
                lax.fori_loop(0, 156, dzc, 0)
                pltpu.sync_copy(dzero.at[pl.ds(0, 160)],
                                deg_sh.at[pl.ds(156 * 640, 160)])

        plsc.subcore_barrier()

        def process(rowbase):
            pltpu.sync_copy(src_hbm.at[pl.ds(rowbase, KB)], sidx)
            pltpu.sync_copy(dst_hbm.at[pl.ds(rowbase, KB)], didx)
            cps = [pltpu.async_copy(z_hbm.at[sidx.at[j]],
                                    rows.at[pl.ds(j * 128, 128)], sem)
                   for j in range(KB)]
            for cp in cps:
                cp.wait()
            scs = [pltpu.async_copy(rows.at[pl.ds(j * 128, 128)],
                                    acc_sh.at[didx.at[j]], sem, add=True)
                   for j in range(KB)]
            if with_deg:
                scs += [pltpu.async_copy(ones_v, deg_sh.at[didx.at[j]],
                                         sem, add=True)
                        for j in range(KB)]
            for cp in scs:
                cp.wait()

        def chunk(it, carry):
            process(wid * RPT + it * KB)
            return carry
        lax.fori_loop(0, NCHUNK, chunk, 0)

        @pl.when(wid < (EROWS - 32 * RPT) // KB)
        def _():
            process(32 * RPT + wid * KB)

        plsc.subcore_barrier()

        def writeout(dst_view, src_view):
            @pl.when(s < 15)
            def _():
                pltpu.sync_copy(src_view.at[pl.ds(s * WCHUNK, WCHUNK)],
                                dst_view.at[pl.ds(s * WCHUNK, WCHUNK)])

            @pl.when(s == 15)
            def _():
                pltpu.sync_copy(
                    src_view.at[pl.ds(15 * WCHUNK, N - 15 * WCHUNK)],
                    dst_view.at[pl.ds(15 * WCHUNK, N - 15 * WCHUNK)])

        writeout(acc_out.at[c], acc_sh)
        if with_deg:
            @pl.when(s < 15)
            def _():
                pltpu.sync_copy(deg_sh.at[pl.ds(s * 6656, 6656)],
                                deg_out.at[c].at[pl.ds(s * 6656, 6656)])

            @pl.when(s == 15)
            def _():
                pltpu.sync_copy(deg_sh.at[pl.ds(15 * 6656, N - 15 * 6656)],
                                deg_out.at[c].at[pl.ds(15 * 6656,
                                                       N - 15 * 6656)])

    return pl.kernel(
        body, mesh=mesh, out_type=out_type, scratch_types=scratch,
        compiler_params=pltpu.CompilerParams(use_tc_tiling_on_sc=False))


_edge_pass_deg = _edge_pass(True)
_edge_pass_nodeg = _edge_pass(False)


def _z1_body(x_ref, w_ref, o_ref):
    o_ref[...] = jnp.dot(x_ref[...], w_ref[...],
                         preferred_element_type=jnp.float32)


def _eye():
    return (lax.broadcasted_iota(jnp.int32, (F, F), 0)
            == lax.broadcasted_iota(jnp.int32, (F, F), 1)).astype(jnp.float32)


def _dg(a, b, dims):
    return lax.dot_general(a, b, (dims, ((), ())),
                           preferred_element_type=jnp.float32)


def _hidden_t(z_ref, acc_ref, deg_ref, b_ref):
    eye = _eye()
    acc = acc_ref[0] + acc_ref[1]
    deg = deg_ref[0, 0, 0] + deg_ref[1, 0, 0]
    acc_t = _dg(eye, acc, ((1,), (1,)))
    z_t = _dg(eye, z_ref[...], ((1,), (1,)))
    mean_t = acc_t / jnp.maximum(deg, 1.0)
    h_t = jnp.where(deg > 0, mean_t, z_t) + b_ref[...]
    return jnp.maximum(h_t, 0.0)


def _mid_body(z_ref, acc_ref, deg_ref, b_ref, w_ref, o_ref):
    h_t = _hidden_t(z_ref, acc_ref, deg_ref, b_ref)
    o_ref[...] = _dg(h_t, w_ref[...], ((0,), (0,)))


def _fin_body(z_ref, acc_ref, deg_ref, b_ref, g_ref,
              wf1_ref, bf1_ref, wf2_ref, bf2_ref, o_ref, hg_scr, cnt_scr):
    i = pl.program_id(0)

    @pl.when(i == 0)
    def _():
        hg_scr[...] = jnp.zeros_like(hg_scr)
        cnt_scr[...] = jnp.zeros_like(cnt_scr)

    h_t = _hidden_t(z_ref, acc_ref, deg_ref, b_ref)
    g = g_ref[0, 0]
    gio = lax.broadcasted_iota(jnp.int32, (NGRAPH, BLK), 0)
    oh = (g == gio).astype(jnp.float32)
    hg_scr[...] += _dg(oh, h_t, ((1,), (1,)))
    cnt_scr[...] += jnp.sum(oh, axis=1, keepdims=True)

    @pl.when(i == NBLK - 1)
    def _():
        hg = hg_scr[...] / jnp.maximum(cnt_scr[...], 1.0)
        a = jnp.dot(hg, wf1_ref[...],
                    preferred_element_type=jnp.float32) + bf1_ref[...]
        p = jnp.dot(a, wf2_ref[...],
                    preferred_element_type=jnp.float32) + bf2_ref[...]
        o_ref[...] = 1.0 / (1.0 + jnp.exp(-p))


def kernel(x, edge_index, graph_ids, W1, b1, W2, b2, Wf1, bf1, Wf2, bf2):
    e2 = edge_index.astype(jnp.int32).reshape(2, EROWS, 128)
    gid = graph_ids.astype(jnp.int32)

    z1 = pl.pallas_call(
        _z1_body,
        grid=(NBLK,),
        in_specs=[pl.BlockSpec((BLK, 20), lambda i: (i, 0)),
                  pl.BlockSpec((20, F), lambda i: (0, 0))],
        out_specs=pl.BlockSpec((BLK, F), lambda i: (i, 0)),
        out_shape=jax.ShapeDtypeStruct((N, F), jnp.float32),
    )(x, W1)

    acc1, deg = _edge_pass_deg(e2, z1)
    deg4 = deg.reshape(2, NBLK, 1, BLK)

    nf_spec = pl.BlockSpec((BLK, F), lambda i: (i, 0))
    acc_spec = pl.BlockSpec((2, BLK, F), lambda i: (0, i, 0))
    deg_spec = pl.BlockSpec((2, 1, 1, BLK), lambda i: (0, i, 0, 0))
    bt_spec = pl.BlockSpec((F, 1), lambda i: (0, 0))

    z2 = pl.pallas_call(
        _mid_body,
        grid=(NBLK,),
        in_specs=[nf_spec, acc_spec, deg_spec, bt_spec,
                  pl.BlockSpec((F, F), lambda i: (0, 0))],
        out_specs=nf_spec,
        out_shape=jax.ShapeDtypeStruct((N, F), jnp.float32),
    )(z1, acc1, deg4, b1.reshape(F, 1), W2)

    (acc2,) = _edge_pass_nodeg(e2, z2)

    out = pl.pallas_call(
        _fin_body,
        grid=(NBLK,),
        in_specs=[nf_spec, acc_spec, deg_spec, bt_spec,
                  pl.BlockSpec((1, 1, BLK), lambda i: (i, 0, 0)),
                  pl.BlockSpec((F, 8), lambda i: (0, 0)),
                  pl.BlockSpec((1, 8), lambda i: (0, 0)),
                  pl.BlockSpec((8, 1), lambda i: (0, 0)),
                  pl.BlockSpec((1, 1), lambda i: (0, 0))],
        out_specs=pl.BlockSpec((NGRAPH, 1), lambda i: (0, 0)),
        out_shape=jax.ShapeDtypeStruct((NGRAPH, 1), jnp.float32),
        scratch_shapes=[pltpu.VMEM((NGRAPH, F), jnp.float32),
                        pltpu.VMEM((NGRAPH, 1), jnp.float32)],
    )(z2, acc2, deg4, b2.reshape(F, 1), gid.reshape(NBLK, 1, BLK),
      Wf1, bf1.reshape(1, 8), Wf2, bf2.reshape(1, 1))

    return out.reshape(NGRAPH)


# speedup vs baseline: 49.8044x; 1.3096x over previous
import functools

import jax
import jax.numpy as jnp
from jax import lax
from jax.experimental import pallas as pl
from jax.experimental.pallas import tpu as pltpu
from jax.experimental.pallas import tpu_sc as plsc

N = 100000
F = 16
NGRAPH = 64
BLK = 2000
NBLK = N // BLK
EROWS = 25000
RPT = 776
KB = 8
NCHUNK = 97
WCHUNK = 6256
ZC = 784


def _edge_pass(with_deg):
    mesh = plsc.VectorSubcoreMesh(core_axis_name="c", subcore_axis_name="s")
    out_type = [jax.ShapeDtypeStruct((2, N, F), jnp.float32)]
    scratch = [
        pltpu.VMEM_SHARED((N, F), jnp.float32),
        pltpu.VMEM((2, KB, 128), jnp.int32),
        pltpu.VMEM((2, KB, 128), jnp.int32),
        pltpu.VMEM((KB * 128, F), jnp.float32),
        pltpu.SemaphoreType.DMA,
        pltpu.SemaphoreType.DMA,
        pltpu.SemaphoreType.DMA,
        pltpu.SemaphoreType.DMA,
    ]
    if with_deg:
        out_type.append(jax.ShapeDtypeStruct((2, N), jnp.float32))
        scratch += [
            pltpu.VMEM_SHARED((N,), jnp.float32),
            pltpu.VMEM((128,), jnp.float32),
            pltpu.VMEM((640,), jnp.float32),
        ]

    def body(*refs):
        if with_deg:
            (e_hbm, z_hbm, acc_out, deg_out,
             acc_sh, sidx, didx, rows, gsA, gsB, ssA, ssB,
             deg_sh, ones_v, dzero) = refs
        else:
            (e_hbm, z_hbm, acc_out,
             acc_sh, sidx, didx, rows, gsA, gsB, ssA, ssB) = refs
        src_hbm = e_hbm.at[0]
        dst_hbm = e_hbm.at[1]
        c = lax.axis_index("c")
        s = lax.axis_index("s")
        wid = c * 16 + s

        def zrow(i, carry):
            rows[i, :] = jnp.zeros((F,), jnp.float32)
            return carry
        lax.fori_loop(0, KB * 128, zrow, 0)

        def zacc(k, carry):
            pltpu.sync_copy(rows,
                            acc_sh.at[pl.ds(s * WCHUNK + k * (KB * 128),
                                            KB * 128)])
            return carry
        lax.fori_loop(0, 6, zacc, 0)

        @pl.when(s < 15)
        def _():
            pltpu.sync_copy(rows.at[pl.ds(0, WCHUNK - 6144)],
                            acc_sh.at[pl.ds(s * WCHUNK + 6144,
                                            WCHUNK - 6144)])

        @pl.when(s == 15)
        def _():
            pltpu.sync_copy(rows.at[pl.ds(0, N - 15 * WCHUNK - 6144)],
                            acc_sh.at[pl.ds(15 * WCHUNK + 6144,
                                            N - 15 * WCHUNK - 6144)])

        if with_deg:
            def o16(i, carry):
                ones_v[pl.ds(i * 16, 16)] = jnp.ones((16,), jnp.float32)
                return carry
            lax.fori_loop(0, 8, o16, 0)

            @pl.when(s == 0)
            def _():
                def dz(i, carry):
                    dzero[pl.ds(i * 16, 16)] = jnp.zeros((16,), jnp.float32)
                    return carry
                lax.fori_loop(0, 40, dz, 0)

                def dzc(k, carry):
                    pltpu.sync_copy(dzero, deg_sh.at[pl.ds(k * 640, 640)])
                    return carry
                lax.fori_loop(0, 156, dzc, 0)
                pltpu.sync_copy(dzero.at[pl.ds(0, 160)],
                                deg_sh.at[pl.ds(156 * 640, 160)])

        plsc.subcore_barrier()

        H = KB // 2

        def load_idx(p, rowbase):
            pltpu.sync_copy(src_hbm.at[pl.ds(rowbase, KB)], sidx.at[p])
            pltpu.sync_copy(dst_hbm.at[pl.ds(rowbase, KB)], didx.at[p])

        def drain_half(sem_, deg_too):
            pltpu.make_async_copy(z_hbm.at[pl.ds(0, H * 128)],
                                  rows.at[pl.ds(0, H * 128)], sem_).wait()
            if deg_too:
                pltpu.make_async_copy(z_hbm.at[pl.ds(0, H * 8)],
                                      rows.at[pl.ds(0, H * 8)], sem_).wait()

        def fire_gathers(p, h, sem_):
            for j in range(H):
                r = h * H + j
                pltpu.async_copy(z_hbm.at[sidx.at[p].at[r]],
                                 rows.at[pl.ds(r * 128, 128)], sem_)

        def fire_scatters(p, h, sem_):
            for j in range(H):
                r = h * H + j
                pltpu.async_copy(rows.at[pl.ds(r * 128, 128)],
                                 acc_sh.at[didx.at[p].at[r]], sem_, add=True)
                if with_deg:
                    pltpu.async_copy(ones_v, deg_sh.at[didx.at[p].at[r]],
                                     sem_, add=True)

        def run_chunk(p, rowbase_next, drain_prev):
            if drain_prev:
                drain_half(ssA, with_deg)
            fire_gathers(p, 0, gsA)
            if drain_prev:
                drain_half(ssB, with_deg)
            fire_gathers(p, 1, gsB)
            if rowbase_next is not None:
                load_idx(1 - p, rowbase_next)
            drain_half(gsA, False)
            fire_scatters(p, 0, ssA)
            drain_half(gsB, False)
            fire_scatters(p, 1, ssB)

        base = wid * RPT
        load_idx(0, base)
        run_chunk(0, base + KB, False)

        def chunk_pair(c2, carry):
            run_chunk(1, base + (2 * c2 + 2) * KB, True)
            run_chunk(0, base + (2 * c2 + 3) * KB, True)
            return carry
        lax.fori_loop(0, 47, chunk_pair, 0)

        run_chunk(1, base + 96 * KB, True)
        run_chunk(0, None, True)
        drain_half(ssA, with_deg)
        drain_half(ssB, with_deg)

        @pl.when(wid < (EROWS - 32 * RPT) // KB)
        def _():
            load_idx(0, 32 * RPT + wid * KB)
            fire_gathers(0, 0, gsA)
            fire_gathers(0, 1, gsB)
            drain_half(gsA, False)
            fire_scatters(0, 0, ssA)
            drain_half(gsB, False)
            fire_scatters(0, 1, ssB)
            drain_half(ssA, with_deg)
            drain_half(ssB, with_deg)

        plsc.subcore_barrier()

        def writeout(dst_view, src_view):
            @pl.when(s < 15)
            def _():
                pltpu.sync_copy(src_view.at[pl.ds(s * WCHUNK, WCHUNK)],
                                dst_view.at[pl.ds(s * WCHUNK, WCHUNK)])

            @pl.when(s == 15)
            def _():
                pltpu.sync_copy(
                    src_view.at[pl.ds(15 * WCHUNK, N - 15 * WCHUNK)],
                    dst_view.at[pl.ds(15 * WCHUNK, N - 15 * WCHUNK)])

        writeout(acc_out.at[c], acc_sh)
        if with_deg:
            @pl.when(s < 15)
            def _():
                pltpu.sync_copy(deg_sh.at[pl.ds(s * 6656, 6656)],
                                deg_out.at[c].at[pl.ds(s * 6656, 6656)])

            @pl.when(s == 15)
            def _():
                pltpu.sync_copy(deg_sh.at[pl.ds(15 * 6656, N - 15 * 6656)],
                                deg_out.at[c].at[pl.ds(15 * 6656,
                                                       N - 15 * 6656)])

    return pl.kernel(
        body, mesh=mesh, out_type=out_type, scratch_types=scratch,
        compiler_params=pltpu.CompilerParams(use_tc_tiling_on_sc=False))


_edge_pass_deg = _edge_pass(True)
_edge_pass_nodeg = _edge_pass(False)


def _z1_body(x_ref, w_ref, o_ref):
    o_ref[...] = jnp.dot(x_ref[...], w_ref[...],
                         preferred_element_type=jnp.float32)


def _eye():
    return (lax.broadcasted_iota(jnp.int32, (F, F), 0)
            == lax.broadcasted_iota(jnp.int32, (F, F), 1)).astype(jnp.float32)


def _dg(a, b, dims):
    return lax.dot_general(a, b, (dims, ((), ())),
                           preferred_element_type=jnp.float32)


def _hidden_t(z_ref, acc_ref, deg_ref, b_ref):
    eye = _eye()
    acc = acc_ref[0] + acc_ref[1]
    deg = deg_ref[0, 0, 0] + deg_ref[1, 0, 0]
    acc_t = _dg(eye, acc, ((1,), (1,)))
    z_t = _dg(eye, z_ref[...], ((1,), (1,)))
    mean_t = acc_t / jnp.maximum(deg, 1.0)
    h_t = jnp.where(deg > 0, mean_t, z_t) + b_ref[...]
    return jnp.maximum(h_t, 0.0)


def _mid_body(z_ref, acc_ref, deg_ref, b_ref, w_ref, o_ref):
    h_t = _hidden_t(z_ref, acc_ref, deg_ref, b_ref)
    o_ref[...] = _dg(h_t, w_ref[...], ((0,), (0,)))


def _fin_body(z_ref, acc_ref, deg_ref, b_ref, g_ref,
              wf1_ref, bf1_ref, wf2_ref, bf2_ref, o_ref, hg_scr, cnt_scr):
    i = pl.program_id(0)

    @pl.when(i == 0)
    def _():
        hg_scr[...] = jnp.zeros_like(hg_scr)
        cnt_scr[...] = jnp.zeros_like(cnt_scr)

    h_t = _hidden_t(z_ref, acc_ref, deg_ref, b_ref)
    g = g_ref[0, 0]
    gio = lax.broadcasted_iota(jnp.int32, (NGRAPH, BLK), 0)
    oh = (g == gio).astype(jnp.float32)
    hg_scr[...] += _dg(oh, h_t, ((1,), (1,)))
    cnt_scr[...] += jnp.sum(oh, axis=1, keepdims=True)

    @pl.when(i == NBLK - 1)
    def _():
        hg = hg_scr[...] / jnp.maximum(cnt_scr[...], 1.0)
        a = jnp.dot(hg, wf1_ref[...],
                    preferred_element_type=jnp.float32) + bf1_ref[...]
        p = jnp.dot(a, wf2_ref[...],
                    preferred_element_type=jnp.float32) + bf2_ref[...]
        o_ref[...] = 1.0 / (1.0 + jnp.exp(-p))


def kernel(x, edge_index, graph_ids, W1, b1, W2, b2, Wf1, bf1, Wf2, bf2):
    e2 = edge_index.astype(jnp.int32).reshape(2, EROWS, 128)
    gid = graph_ids.astype(jnp.int32)

    z1 = pl.pallas_call(
        _z1_body,
        grid=(NBLK,),
        in_specs=[pl.BlockSpec((BLK, 20), lambda i: (i, 0)),
                  pl.BlockSpec((20, F), lambda i: (0, 0))],
        out_specs=pl.BlockSpec((BLK, F), lambda i: (i, 0)),
        out_shape=jax.ShapeDtypeStruct((N, F), jnp.float32),
    )(x, W1)

    acc1, deg = _edge_pass_deg(e2, z1)
    deg4 = deg.reshape(2, NBLK, 1, BLK)

    nf_spec = pl.BlockSpec((BLK, F), lambda i: (i, 0))
    acc_spec = pl.BlockSpec((2, BLK, F), lambda i: (0, i, 0))
    deg_spec = pl.BlockSpec((2, 1, 1, BLK), lambda i: (0, i, 0, 0))
    bt_spec = pl.BlockSpec((F, 1), lambda i: (0, 0))

    z2 = pl.pallas_call(
        _mid_body,
        grid=(NBLK,),
        in_specs=[nf_spec, acc_spec, deg_spec, bt_spec,
                  pl.BlockSpec((F, F), lambda i: (0, 0))],
        out_specs=nf_spec,
        out_shape=jax.ShapeDtypeStruct((N, F), jnp.float32),
    )(z1, acc1, deg4, b1.reshape(F, 1), W2)

    (acc2,) = _edge_pass_nodeg(e2, z2)

    out = pl.pallas_call(
        _fin_body,
        grid=(NBLK,),
        in_specs=[nf_spec, acc_spec, deg_spec, bt_spec,
                  pl.BlockSpec((1, 1, BLK), lambda i: (i, 0, 0)),
                  pl.BlockSpec((F, 8), lambda i: (0, 0)),
                  pl.BlockSpec((1, 8), lambda i: (0, 0)),
                  pl.BlockSpec((8, 1), lambda i: (0, 0)),
                  pl.BlockSpec((1, 1), lambda i: (0, 0))],
        out_specs=pl.BlockSpec((NGRAPH, 1), lambda i: (0, 0)),
        out_shape=jax.ShapeDtypeStruct((NGRAPH, 1), jnp.float32),
        scratch_shapes=[pltpu.VMEM((NGRAPH, F), jnp.float32),
                        pltpu.VMEM((NGRAPH, 1), jnp.float32)],
    )(z2, acc2, deg4, b2.reshape(F, 1), gid.reshape(NBLK, 1, BLK),
      Wf1, bf1.reshape(1, 8), Wf2, bf2.reshape(1, 1))

    return out.reshape(NGRAPH)


# speedup vs baseline: 71.5262x; 1.4361x over previous
import functools

import jax
import jax.numpy as jnp
from jax import lax
from jax.experimental import pallas as pl
from jax.experimental.pallas import tpu as pltpu
from jax.experimental.pallas import tpu_sc as plsc

N = 100000
F = 16
NGRAPH = 64
BLK = 2000
NBLK = N // BLK
EROWS = 25000
RPT = 776
KB = 8
NCHUNK = 97
WCHUNK = 6256
ZC = 784


def _edge_pass(with_deg):
    mesh = plsc.VectorSubcoreMesh(core_axis_name="c", subcore_axis_name="s")
    out_type = [jax.ShapeDtypeStruct((2, N, F), jnp.float32)]
    scratch = [
        pltpu.VMEM_SHARED((N, F), jnp.float32),
        pltpu.VMEM((2, KB, 128), jnp.int32),
        pltpu.VMEM((2, KB, 128), jnp.int32),
        pltpu.VMEM((KB * 128, F), jnp.float32),
        pltpu.SemaphoreType.DMA,
        pltpu.SemaphoreType.DMA,
        pltpu.SemaphoreType.DMA,
        pltpu.SemaphoreType.DMA,
    ]
    if with_deg:
        out_type.append(jax.ShapeDtypeStruct((2, N), jnp.float32))
        scratch += [
            pltpu.VMEM_SHARED((N,), jnp.float32),
            pltpu.VMEM((128,), jnp.float32),
            pltpu.VMEM((640,), jnp.float32),
        ]

    def body(*refs):
        if with_deg:
            (e_hbm, z_hbm, acc_out, deg_out,
             acc_sh, sidx, didx, rows, gsA, gsB, ssA, ssB,
             deg_sh, ones_v, dzero) = refs
        else:
            (e_hbm, z_hbm, acc_out,
             acc_sh, sidx, didx, rows, gsA, gsB, ssA, ssB) = refs
        src_hbm = e_hbm.at[0]
        dst_hbm = e_hbm.at[1]
        c = lax.axis_index("c")
        s = lax.axis_index("s")
        wid = c * 16 + s

        def zrow(i, carry):
            rows[i, :] = jnp.zeros((F,), jnp.float32)
            return carry
        lax.fori_loop(0, KB * 128, zrow, 0)

        def zacc(k, carry):
            pltpu.sync_copy(rows,
                            acc_sh.at[pl.ds(s * WCHUNK + k * (KB * 128),
                                            KB * 128)])
            return carry
        lax.fori_loop(0, 6, zacc, 0)

        @pl.when(s < 15)
        def _():
            pltpu.sync_copy(rows.at[pl.ds(0, WCHUNK - 6144)],
                            acc_sh.at[pl.ds(s * WCHUNK + 6144,
                                            WCHUNK - 6144)])

        @pl.when(s == 15)
        def _():
            pltpu.sync_copy(rows.at[pl.ds(0, N - 15 * WCHUNK - 6144)],
                            acc_sh.at[pl.ds(15 * WCHUNK + 6144,
                                            N - 15 * WCHUNK - 6144)])

        if with_deg:
            def o16(i, carry):
                ones_v[pl.ds(i * 16, 16)] = jnp.ones((16,), jnp.float32)
                return carry
            lax.fori_loop(0, 8, o16, 0)

            @pl.when(s == 0)
            def _():
                def dz(i, carry):
                    dzero[pl.ds(i * 16, 16)] = jnp.zeros((16,), jnp.float32)
                    return carry
                lax.fori_loop(0, 40, dz, 0)

                def dzc(k, carry):
                    pltpu.sync_copy(dzero, deg_sh.at[pl.ds(k * 640, 640)])
                    return carry
                lax.fori_loop(0, 156, dzc, 0)
                pltpu.sync_copy(dzero.at[pl.ds(0, 160)],
                                deg_sh.at[pl.ds(156 * 640, 160)])

        plsc.subcore_barrier()

        H = KB // 2

        def load_idx(p, rowbase):
            pltpu.sync_copy(src_hbm.at[pl.ds(rowbase, KB)], sidx.at[p])
            pltpu.sync_copy(dst_hbm.at[pl.ds(rowbase, KB)], didx.at[p])

        def drain_half(sem_, deg_too):
            pltpu.make_async_copy(z_hbm.at[pl.ds(0, H * 128)],
                                  rows.at[pl.ds(0, H * 128)], sem_).wait()
            if deg_too:
                pltpu.make_async_copy(z_hbm.at[pl.ds(0, H * 8)],
                                      rows.at[pl.ds(0, H * 8)], sem_).wait()

        def fire_gathers(p, h, sem_):
            for j in range(H):
                r = h * H + j
                pltpu.async_copy(z_hbm.at[sidx.at[p].at[r]],
                                 rows.at[pl.ds(r * 128, 128)], sem_)

        def fire_scatters(p, h, sem_):
            for j in range(H):
                r = h * H + j
                pltpu.async_copy(rows.at[pl.ds(r * 128, 128)],
                                 acc_sh.at[didx.at[p].at[r]], sem_, add=True)
                if with_deg:
                    pltpu.async_copy(ones_v, deg_sh.at[didx.at[p].at[r]],
                                     sem_, add=True)

        def run_chunk(p, rowbase_next, drain_prev):
            if drain_prev:
                drain_half(ssA, with_deg)
            fire_gathers(p, 0, gsA)
            if drain_prev:
                drain_half(ssB, with_deg)
            fire_gathers(p, 1, gsB)
            if rowbase_next is not None:
                load_idx(1 - p, rowbase_next)
            drain_half(gsA, False)
            fire_scatters(p, 0, ssA)
            drain_half(gsB, False)
            fire_scatters(p, 1, ssB)

        base = wid * RPT
        load_idx(0, base)
        run_chunk(0, base + KB, False)

        def chunk_pair(c2, carry):
            run_chunk(1, base + (2 * c2 + 2) * KB, True)
            run_chunk(0, base + (2 * c2 + 3) * KB, True)
            return carry
        lax.fori_loop(0, 47, chunk_pair, 0)

        run_chunk(1, base + 96 * KB, True)
        run_chunk(0, None, True)
        drain_half(ssA, with_deg)
        drain_half(ssB, with_deg)

        @pl.when(wid < (EROWS - 32 * RPT) // KB)
        def _():
            load_idx(0, 32 * RPT + wid * KB)
            fire_gathers(0, 0, gsA)
            fire_gathers(0, 1, gsB)
            drain_half(gsA, False)
            fire_scatters(0, 0, ssA)
            drain_half(gsB, False)
            fire_scatters(0, 1, ssB)
            drain_half(ssA, with_deg)
            drain_half(ssB, with_deg)

        plsc.subcore_barrier()

        def writeout(dst_view, src_view):
            @pl.when(s < 15)
            def _():
                pltpu.sync_copy(src_view.at[pl.ds(s * WCHUNK, WCHUNK)],
                                dst_view.at[pl.ds(s * WCHUNK, WCHUNK)])

            @pl.when(s == 15)
            def _():
                pltpu.sync_copy(
                    src_view.at[pl.ds(15 * WCHUNK, N - 15 * WCHUNK)],
                    dst_view.at[pl.ds(15 * WCHUNK, N - 15 * WCHUNK)])

        writeout(acc_out.at[c], acc_sh)
        if with_deg:
            @pl.when(s < 15)
            def _():
                pltpu.sync_copy(deg_sh.at[pl.ds(s * 6656, 6656)],
                                deg_out.at[c].at[pl.ds(s * 6656, 6656)])

            @pl.when(s == 15)
            def _():
                pltpu.sync_copy(deg_sh.at[pl.ds(15 * 6656, N - 15 * 6656)],
                                deg_out.at[c].at[pl.ds(15 * 6656,
                                                       N - 15 * 6656)])

    return pl.kernel(
        body, mesh=mesh, out_type=out_type, scratch_types=scratch,
        compiler_params=pltpu.CompilerParams(use_tc_tiling_on_sc=False))


_edge_pass_deg = _edge_pass(True)
_edge_pass_nodeg = _edge_pass(False)


def _z1_body(x_ref, w_ref, o_ref):
    o_ref[...] = jnp.dot(x_ref[...], w_ref[...],
                         preferred_element_type=jnp.float32)


NR = N // 8


def _dg(a, b, dims):
    return lax.dot_general(a, b, (dims, ((), ())),
                           preferred_element_type=jnp.float32)


def _z1_packed_body(x_ref, w_ref, o_ref):
    o_ref[...] = _dg(x_ref[...], w_ref[...], ((1,), (0,)))


def _hidden_p(z_ref, acc_ref, deg_ref, r8_ref, b_ref):
    acc = acc_ref[0] + acc_ref[1]
    degb = deg_ref[0] + deg_ref[1]
    recip = 1.0 / jnp.maximum(degb, 1.0)
    mask = (degb > 0).astype(jnp.float32)
    recipx = _dg(recip, r8_ref[...], ((1,), (0,)))
    maskx = _dg(mask, r8_ref[...], ((1,), (0,)))
    h = maskx * (acc * recipx) + (1.0 - maskx) * z_ref[...] + b_ref[...]
    return jnp.maximum(h, 0.0)


def _mid_body(z_ref, acc_ref, deg_ref, r8_ref, b_ref, w_ref, o_ref):
    h = _hidden_p(z_ref, acc_ref, deg_ref, r8_ref, b_ref)
    o_ref[...] = _dg(h, w_ref[...], ((1,), (0,)))


def _fin_body(z_ref, acc_ref, deg_ref, r8_ref, b_ref, g_ref,
              wf1_ref, bf1_ref, wf2_ref, bf2_ref, o_ref):
    h = _hidden_p(z_ref, acc_ref, deg_ref, r8_ref, b_ref)
    gidb = g_ref[...].astype(jnp.float32)
    eye8 = (lax.broadcasted_iota(jnp.int32, (8, 8), 0)
            == lax.broadcasted_iota(jnp.int32, (8, 8), 1)).astype(jnp.float32)
    g_t = _dg(eye8, gidb, ((1,), (1,))).astype(jnp.int32)
    giota = lax.broadcasted_iota(jnp.int32, (NGRAPH, NR), 0)
    ones_col = jnp.ones((NR, 1), jnp.float32)
    hg = jnp.zeros((NGRAPH, F), jnp.float32)
    cnt = jnp.zeros((NGRAPH, 1), jnp.float32)
    for r in range(8):
        oh_r = (g_t[r] == giota).astype(jnp.float32)
        t_r = _dg(oh_r, h, ((1,), (0,)))
        hg = hg + t_r[:, r * F:(r + 1) * F]
        cnt = cnt + _dg(oh_r, ones_col, ((1,), (0,)))
    hgm = hg / jnp.maximum(cnt, 1.0)
    a = jnp.dot(hgm, wf1_ref[...],
                preferred_element_type=jnp.float32) + bf1_ref[...]
    p = jnp.dot(a, wf2_ref[...],
                preferred_element_type=jnp.float32) + bf2_ref[...]
    o_ref[...] = 1.0 / (1.0 + jnp.exp(-p))


def kernel(x, edge_index, graph_ids, W1, b1, W2, b2, Wf1, bf1, Wf2, bf2):
    e2 = edge_index.astype(jnp.int32).reshape(2, EROWS, 128)
    gidp = graph_ids.astype(jnp.int32).reshape(NR, 8)

    eye8 = jnp.eye(8, dtype=jnp.float32)
    w1bd = jnp.kron(eye8, W1)
    w2bd = jnp.kron(eye8, W2)
    r8 = jnp.kron(eye8, jnp.ones((1, F), jnp.float32))
    b1row = jnp.tile(b1, 8).reshape(1, 128)
    b2row = jnp.tile(b2, 8).reshape(1, 128)
    z1p = pl.pallas_call(
        _z1_packed_body,
        grid=(1,),
        in_specs=[pl.BlockSpec((NR, 160), lambda i: (0, 0)),
                  pl.BlockSpec((160, 128), lambda i: (0, 0))],
        out_specs=pl.BlockSpec((NR, 128), lambda i: (0, 0)),
        out_shape=jax.ShapeDtypeStruct((NR, 128), jnp.float32),
    )(x.reshape(NR, 160), w1bd)

    acc1, deg = _edge_pass_deg(e2, z1p.reshape(N, F))
    degp = deg.reshape(2, NR, 8)

    pk_spec = pl.BlockSpec((NR, 128), lambda i: (0, 0))
    acc_spec = pl.BlockSpec((2, NR, 128), lambda i: (0, 0, 0))
    deg_spec = pl.BlockSpec((2, NR, 8), lambda i: (0, 0, 0))
    r8_spec = pl.BlockSpec((8, 128), lambda i: (0, 0))
    b_spec = pl.BlockSpec((1, 128), lambda i: (0, 0))

    z2p = pl.pallas_call(
        _mid_body,
        grid=(1,),
        in_specs=[pk_spec, acc_spec, deg_spec, r8_spec, b_spec,
                  pl.BlockSpec((128, 128), lambda i: (0, 0))],
        out_specs=pk_spec,
        out_shape=jax.ShapeDtypeStruct((NR, 128), jnp.float32),
    )(z1p, acc1.reshape(2, NR, 128), degp, r8, b1row, w2bd)

    (acc2,) = _edge_pass_nodeg(e2, z2p.reshape(N, F))

    out = pl.pallas_call(
        _fin_body,
        grid=(1,),
        in_specs=[pk_spec, acc_spec, deg_spec, r8_spec, b_spec,
                  pl.BlockSpec((NR, 8), lambda i: (0, 0)),
                  pl.BlockSpec((F, 8), lambda i: (0, 0)),
                  pl.BlockSpec((1, 8), lambda i: (0, 0)),
                  pl.BlockSpec((8, 1), lambda i: (0, 0)),
                  pl.BlockSpec((1, 1), lambda i: (0, 0))],
        out_specs=pl.BlockSpec((NGRAPH, 1), lambda i: (0, 0)),
        out_shape=jax.ShapeDtypeStruct((NGRAPH, 1), jnp.float32),
    )(z2p, acc2.reshape(2, NR, 128), degp, r8, b2row, gidp,
      Wf1, bf1.reshape(1, 8), Wf2, bf2.reshape(1, 1))

    return out.reshape(NGRAPH)
